# Initial kernel scaffold; baseline (speedup 1.0000x reference)
import jax
import jax.numpy as jnp
from jax.experimental import pallas as pl


def kernel(x, tables):
    raise NotImplementedError("write your pallas kernel here")


# speedup vs baseline: 7.4429x; 7.4429x over previous
import functools

import jax
import jax.numpy as jnp
from jax import lax
from jax.experimental import pallas as pl
from jax.experimental.pallas import tpu as pltpu
from jax.experimental.pallas import tpu_sc as plsc

D = 256
N = 100000
NCORES, NSUB = 2, 16
NW = NCORES * NSUB
ROWS_W = 3200
CHUNK = 64
NCHUNK = ROWS_W // CHUNK

DIMS_A = (119, 2, 2)
DIMS_B = (4, 12, 12)
DIMS_C = (10, 6, 6)
PAD_A = 480
OFF_B = 480
OFF_C = 480 + 576
TROWS = 480 + 576 + 360


def _build_body(t0, t1, t2, t3, t4, t5, t6, t7, t8, out_ref):
    def combo(tabs, dims, n_pad):
        acc = jnp.zeros((n_pad, D), jnp.float32)
        st = 1
        for d in dims:
            st *= d
        for t, d in zip(tabs, dims):
            st //= d
            r = (lax.broadcasted_iota(jnp.int32, (n_pad, d), 0) // st) % d
            c = lax.broadcasted_iota(jnp.int32, (n_pad, d), 1)
            oh = (r == c).astype(jnp.float32)
            acc = acc + jnp.dot(oh, t, preferred_element_type=jnp.float32,
                                precision=lax.Precision.HIGHEST)
        return acc

    out_ref[0:PAD_A, :] = combo([t0[...], t7[...], t8[...]], DIMS_A, PAD_A)
    out_ref[OFF_B:OFF_C, :] = combo([t1[...], t2[...], t3[...]], DIMS_B, 576)
    out_ref[OFF_C:TROWS, :] = combo([t4[...], t5[...], t6[...]], DIMS_C, 360)


def _build_tables(tables):
    return pl.pallas_call(
        _build_body,
        out_shape=jax.ShapeDtypeStruct((TROWS, D), jnp.float32),
    )(*tables)


_MESH = plsc.VectorSubcoreMesh(core_axis_name="c", subcore_axis_name="s")


@functools.partial(
    pl.kernel,
    out_type=jax.ShapeDtypeStruct((N, D), jnp.float32),
    mesh=_MESH,
    scratch_types=[
        pltpu.VMEM((9, CHUNK), jnp.int32),
        pltpu.VMEM((CHUNK,), jnp.int32),
        pltpu.VMEM((CHUNK,), jnp.int32),
        pltpu.VMEM((CHUNK,), jnp.int32),
        pltpu.VMEM((CHUNK, D), jnp.float32),
        pltpu.VMEM((CHUNK, D), jnp.float32),
        pltpu.VMEM((CHUNK, D), jnp.float32),
        pltpu.VMEM((CHUNK, D), jnp.float32),
        pltpu.SemaphoreType.DMA,
    ],
)
def _sc_gather_sum(xT, btab, out, xv, idA, idB, idC, rA, rB, rC, outv, sem):
    wid = lax.axis_index("s") * NCORES + lax.axis_index("c")
    base = jnp.minimum(wid * ROWS_W, N - ROWS_W)

    def chunk_body(j, _):
        rbase = base + j * CHUNK
        hs = [
            pltpu.async_copy(xT.at[pl.ds(f * N + rbase, CHUNK)], xv.at[f], sem)
            for f in range(9)
        ]
        for h in hs:
            h.wait()

        def id_body(s, _):
            sl = pl.ds(s * 16, 16)
            idA[sl] = xv[0, sl] * 4 + xv[7, sl] * 2 + xv[8, sl]
            idB[sl] = xv[1, sl] * 144 + xv[2, sl] * 12 + xv[3, sl] + OFF_B
            idC[sl] = xv[4, sl] * 36 + xv[5, sl] * 6 + xv[6, sl] + OFF_C
            return 0

        lax.fori_loop(0, CHUNK // 16, id_body, 0)

        ca = pltpu.async_copy(btab.at[idA], rA, sem)
        cb = pltpu.async_copy(btab.at[idB], rB, sem)
        cc = pltpu.async_copy(btab.at[idC], rC, sem)
        ca.wait()
        cb.wait()
        cc.wait()

        def acc_body(c, _):
            for g in range(D // 16):
                sl = pl.ds(g * 16, 16)
                outv[c, sl] = rA[c, sl] + rB[c, sl] + rC[c, sl]
            return 0

        lax.fori_loop(0, CHUNK, acc_body, 0)
        pltpu.sync_copy(outv, out.at[pl.ds(rbase, CHUNK)])
        return 0

    lax.fori_loop(0, NCHUNK, chunk_body, 0)


def kernel(x, tables):
    btab = _build_tables(tables)
    xT = jnp.transpose(x).reshape(-1)
    return _sc_gather_sum(xT, btab)


# speedup vs baseline: 10.2928x; 1.3829x over previous
import functools

import jax
import jax.numpy as jnp
from jax import lax
from jax.experimental import pallas as pl
from jax.experimental.pallas import tpu as pltpu
from jax.experimental.pallas import tpu_sc as plsc

D = 256
N = 100000
NCORES, NSUB = 2, 16
NW = NCORES * NSUB
ROWS_W = 3200
CHUNK = 64
NCHUNK = ROWS_W // CHUNK

DIMS_A = (119, 2, 2)
DIMS_B = (4, 12, 12)
DIMS_C = (10, 6, 6)
PAD_A = 480
OFF_B = 480
OFF_C = 480 + 576
TROWS = 480 + 576 + 360


def _build_body(t0, t1, t2, t3, t4, t5, t6, t7, t8, out_ref):
    def combo(tabs, dims, n_pad):
        acc = jnp.zeros((n_pad, D), jnp.float32)
        st = 1
        for d in dims:
            st *= d
        for t, d in zip(tabs, dims):
            st //= d
            r = (lax.broadcasted_iota(jnp.int32, (n_pad, d), 0) // st) % d
            c = lax.broadcasted_iota(jnp.int32, (n_pad, d), 1)
            oh = (r == c).astype(jnp.float32)
            acc = acc + jnp.dot(oh, t, preferred_element_type=jnp.float32,
                                precision=lax.Precision.HIGHEST)
        return acc

    out_ref[0:PAD_A, :] = combo([t0[...], t7[...], t8[...]], DIMS_A, PAD_A)
    out_ref[OFF_B:OFF_C, :] = combo([t1[...], t2[...], t3[...]], DIMS_B, 576)
    out_ref[OFF_C:TROWS, :] = combo([t4[...], t5[...], t6[...]], DIMS_C, 360)


def _build_tables(tables):
    return pl.pallas_call(
        _build_body,
        out_shape=jax.ShapeDtypeStruct((TROWS, D), jnp.float32),
    )(*tables)


_MESH = plsc.VectorSubcoreMesh(core_axis_name="c", subcore_axis_name="s")


@functools.partial(
    pl.kernel,
    out_type=jax.ShapeDtypeStruct((N, D), jnp.float32),
    mesh=_MESH,
    scratch_types=[
        [pltpu.VMEM((ROWS_W,), jnp.int32) for _ in range(9)],
        pltpu.VMEM((CHUNK,), jnp.int32),
        pltpu.VMEM((CHUNK,), jnp.int32),
        pltpu.VMEM((CHUNK,), jnp.int32),
        pltpu.VMEM((CHUNK,), jnp.int32),
        pltpu.VMEM((CHUNK,), jnp.int32),
        pltpu.VMEM((CHUNK,), jnp.int32),
        pltpu.VMEM((CHUNK, D), jnp.float32),
        pltpu.VMEM((CHUNK, D), jnp.float32),
        pltpu.VMEM((CHUNK, D), jnp.float32),
        pltpu.VMEM((CHUNK, D), jnp.float32),
        pltpu.VMEM((CHUNK, D), jnp.float32),
        pltpu.VMEM((CHUNK, D), jnp.float32),
        pltpu.SemaphoreType.DMA,
        pltpu.SemaphoreType.DMA,
        pltpu.SemaphoreType.DMA,
        pltpu.SemaphoreType.DMA,
        pltpu.SemaphoreType.DMA,
    ],
)
def _sc_gather_sum(xT, btab, out, xv,
                   idA0, idB0, idC0, idA1, idB1, idC1,
                   rA0, rB0, rC0, rA1, rB1, rC1,
                   xsem, gsem0, gsem1, osem0, osem1):
    wid = lax.axis_index("s") * NCORES + lax.axis_index("c")
    base = jnp.minimum(wid * ROWS_W, N - ROWS_W)

    ids = ((idA0, idB0, idC0), (idA1, idB1, idC1))
    rbufs = ((rA0, rB0, rC0), (rA1, rB1, rC1))
    gsems = (gsem0, gsem1)
    osems = (osem0, osem1)

    def compute_ids(j, s):
        idA, idB, idC = ids[s]

        def id_body(t, _):
            o = pl.ds(j * CHUNK + t * 16, 16)
            sl = pl.ds(t * 16, 16)
            idA[sl] = xv[0][o] * 4 + xv[7][o] * 2 + xv[8][o]
            idB[sl] = xv[1][o] * 144 + xv[2][o] * 12 + xv[3][o] + OFF_B
            idC[sl] = xv[4][o] * 36 + xv[5][o] * 6 + xv[6][o] + OFF_C
            return 0

        lax.fori_loop(0, CHUNK // 16, id_body, 0)

    def fire_gathers(s):
        for ix, rb in zip(ids[s], rbufs[s]):
            pltpu.async_copy(btab.at[ix], rb, gsems[s])

    def wait_gathers(s):
        for ix, rb in zip(ids[s], rbufs[s]):
            pltpu.make_async_copy(btab.at[ix], rb, gsems[s]).wait()

    def accumulate(s):
        rA, rB, rC = rbufs[s]

        def acc_body(c, _):
            for g in range(D // 16):
                sl = pl.ds(g * 16, 16)
                rA[c, sl] = rA[c, sl] + rB[c, sl] + rC[c, sl]
            return 0

        lax.fori_loop(0, CHUNK, acc_body, 0)

    def fire_write(j, s):
        pltpu.async_copy(rbufs[s][0], out.at[pl.ds(base + j * CHUNK, CHUNK)],
                         osems[s])

    def wait_write(j, s):
        pltpu.make_async_copy(rbufs[s][0],
                              out.at[pl.ds(base + j * CHUNK, CHUNK)],
                              osems[s]).wait()

    xcopies = [
        pltpu.async_copy(xT.at[pl.ds(f * N + base, ROWS_W)], xv[f], xsem)
        for f in range(9)
    ]
    for h in xcopies:
        h.wait()
    compute_ids(0, 0)
    fire_gathers(0)

    compute_ids(1, 1)
    fire_gathers(1)
    wait_gathers(0)
    accumulate(0)
    fire_write(0, 0)

    def half(j, s):
        s2 = 1 - s
        wait_write(j - 1, s2)
        compute_ids(j + 1, s2)
        fire_gathers(s2)
        wait_gathers(s)
        accumulate(s)
        fire_write(j, s)

    def pair_body(k, _):
        j = 1 + 2 * k
        half(j, 1)
        half(j + 1, 0)
        return 0

    lax.fori_loop(0, (NCHUNK - 2) // 2, pair_body, 0)

    wait_write(NCHUNK - 2, 0)
    wait_gathers(1)
    accumulate(1)
    fire_write(NCHUNK - 1, 1)
    wait_write(NCHUNK - 1, 1)


def kernel(x, tables):
    btab = _build_tables(tables)
    xT = jnp.transpose(x).reshape(-1)
    return _sc_gather_sum(xT, btab)
